# Initial kernel scaffold; baseline (speedup 1.0000x reference)
import jax
import jax.numpy as jnp
from jax.experimental import pallas as pl


def kernel(features, weight_theta, weight_bias, bias, neighborhood, positions):
    raise NotImplementedError("write your pallas kernel here")


# speedup vs baseline: 445.7808x; 445.7808x over previous
import functools

import jax
import jax.numpy as jnp
from jax import lax
from jax.experimental import pallas as pl
from jax.experimental.pallas import tpu as pltpu
from jax.experimental.pallas import tpu_sc as plsc

B, C_IN, C_OUT, N, K, D_POS = 1, 128, 128, 10000, 32, 3
NC, NS, L = 2, 16, 16
NW = NC * NS
N_PAD = 10240
P_PER = N_PAD // NW
ROW = C_IN + D_POS * L
CB = C_IN // L
AW = C_IN * (1 + D_POS)
FLUSH = 32
BN = 512


def _sc_body(table_hbm, nbr_hbm, a_hbm, nbr_v, rows_v, abuf_v, sem0, sem1):
    wid = lax.axis_index("s") * NC + lax.axis_index("c")
    base = wid * P_PER
    pltpu.sync_copy(nbr_hbm.at[pl.ds(base, P_PER + 8)], nbr_v)

    def _gather_start(i, buf, sem):
        return pltpu.async_copy(table_hbm.at[nbr_v.at[i]], rows_v.at[buf], sem)

    def _gather_wait(i, buf, sem):
        pltpu.make_async_copy(table_hbm.at[nbr_v.at[i]], rows_v.at[buf], sem).wait()

    def _compute_point(buf, row):
        zero = jnp.zeros((L,), jnp.float32)
        s_acc = [zero] * CB
        u_acc = [[zero] * CB for _ in range(D_POS)]
        for k in range(K):
            p = [rows_v[buf, k, pl.ds(C_IN + d * L, L)] for d in range(D_POS)]
            for cb in range(CB):
                f = rows_v[buf, k, pl.ds(cb * L, L)]
                s_acc[cb] = s_acc[cb] + f
                for d in range(D_POS):
                    u_acc[d][cb] = u_acc[d][cb] + p[d] * f
        for cb in range(CB):
            abuf_v[row, pl.ds(cb * L, L)] = s_acc[cb]
        for d in range(D_POS):
            for cb in range(CB):
                abuf_v[row, pl.ds(C_IN + d * C_IN + cb * L, L)] = u_acc[d][cb]

    _gather_start(0, 0, sem0)

    def _block(blk, _):
        def _pair(jj, _):
            i0 = blk * FLUSH + 2 * jj
            _gather_wait(i0, 0, sem0)
            _gather_start(i0 + 1, 1, sem1)
            _compute_point(0, 2 * jj)
            _gather_wait(i0 + 1, 1, sem1)
            _gather_start(i0 + 2, 0, sem0)
            _compute_point(1, 2 * jj + 1)
            return 0
        lax.fori_loop(0, FLUSH // 2, _pair, 0)
        pltpu.sync_copy(abuf_v, a_hbm.at[pl.ds(base + blk * FLUSH, FLUSH)])
        return 0

    lax.fori_loop(0, P_PER // FLUSH, _block, 0)
    _gather_wait(P_PER, 0, sem0)


def _sc_segment_sums(table, nbr_pad):
    mesh = plsc.VectorSubcoreMesh(core_axis_name="c", subcore_axis_name="s",
                                  num_cores=NC, num_subcores=NS)
    return pl.kernel(
        _sc_body,
        out_type=jax.ShapeDtypeStruct((N_PAD, AW), jnp.float32),
        mesh=mesh,
        compiler_params=pltpu.CompilerParams(use_tc_tiling_on_sc=False),
        scratch_types=[
            pltpu.VMEM((P_PER + 8, K), jnp.int32),
            pltpu.VMEM((2, K, ROW), jnp.float32),
            pltpu.VMEM((FLUSH, AW), jnp.float32),
            pltpu.SemaphoreType.DMA,
            pltpu.SemaphoreType.DMA,
        ],
    )(table, nbr_pad)


def _tc_body(a_ref, w_ref, th_ref, p_ref, b_ref, o_ref):
    a = a_ref[...]
    y = jnp.dot(a, w_ref[...], preferred_element_type=jnp.float32)
    z = jnp.dot(a[:, :C_IN], th_ref[...], preferred_element_type=jnp.float32)
    for d in range(D_POS):
        y = y - p_ref[:, d:d + 1] * z[:, d * C_OUT:(d + 1) * C_OUT]
    o_ref[...] = y + b_ref[0:1, :]


def _tc_combine(a, w_full, th_cat, p_pad, bias2):
    return pl.pallas_call(
        _tc_body,
        grid=(N_PAD // BN,),
        in_specs=[
            pl.BlockSpec((BN, AW), lambda i: (i, 0)),
            pl.BlockSpec((AW, C_OUT), lambda i: (0, 0)),
            pl.BlockSpec((C_IN, D_POS * C_OUT), lambda i: (0, 0)),
            pl.BlockSpec((BN, 128), lambda i: (i, 0)),
            pl.BlockSpec((8, C_OUT), lambda i: (0, 0)),
        ],
        out_specs=pl.BlockSpec((BN, C_OUT), lambda i: (i, 0)),
        out_shape=jax.ShapeDtypeStruct((N_PAD, C_OUT), jnp.float32),
    )(a, w_full, th_cat, p_pad, bias2)


def kernel(features, weight_theta, weight_bias, bias, neighborhood, positions):
    fT = features[0].T
    pT = positions[0].T
    table = jnp.concatenate([fT, jnp.repeat(pT, L, axis=1)], axis=1)
    nbr_pad = jnp.zeros((N_PAD + 8, K), jnp.int32).at[:N].set(neighborhood[0].T)

    a = _sc_segment_sums(table, nbr_pad)

    w_full = jnp.concatenate([weight_bias, weight_theta.reshape(D_POS * C_IN, C_OUT)], axis=0)
    th_cat = jnp.transpose(weight_theta, (1, 0, 2)).reshape(C_IN, D_POS * C_OUT)
    p_pad = jnp.zeros((N_PAD, 128), jnp.float32).at[:N, :D_POS].set(pT)
    bias2 = jnp.broadcast_to(bias[None, :], (8, C_OUT))

    out_t = _tc_combine(a, w_full, th_cat, p_pad, bias2)
    return out_t[:N].T[None]


# speedup vs baseline: 543.9122x; 1.2201x over previous
import functools

import jax
import jax.numpy as jnp
from jax import lax
from jax.experimental import pallas as pl
from jax.experimental.pallas import tpu as pltpu
from jax.experimental.pallas import tpu_sc as plsc

B, C_IN, C_OUT, N, K, D_POS = 1, 128, 128, 10000, 32, 3
NC, NS, L = 2, 16, 16
NW = NC * NS
N_PAD = 10240
P_PER = N_PAD // NW
ROW = C_IN + D_POS * L
CB = C_IN // L
AW = C_IN * (1 + D_POS)
GP = 128 // K
G_PER = P_PER // GP
FLUSH = 32
GF = FLUSH // GP
BN = 512


def _sc_body(table_hbm, nbr_hbm, a_hbm, nbr_v, rows_v, abuf_v, sem0, sem1):
    wid = lax.axis_index("s") * NC + lax.axis_index("c")
    base_g = wid * G_PER
    base_p = wid * P_PER
    pltpu.sync_copy(nbr_hbm.at[pl.ds(base_g, G_PER + 8)], nbr_v)

    def _gather_start(g, buf, sem):
        return pltpu.async_copy(table_hbm.at[nbr_v.at[g]], rows_v.at[buf], sem)

    def _gather_wait(g, buf, sem):
        pltpu.make_async_copy(table_hbm.at[nbr_v.at[g]], rows_v.at[buf], sem).wait()

    def _compute_group(buf, row0):
        for q in range(GP):
            zero = jnp.zeros((L,), jnp.float32)
            s_acc = [zero] * CB
            u_acc = [[zero] * CB for _ in range(D_POS)]
            for k in range(K):
                r = q * K + k
                p = [rows_v[buf, r, pl.ds(C_IN + d * L, L)] for d in range(D_POS)]
                for cb in range(CB):
                    f = rows_v[buf, r, pl.ds(cb * L, L)]
                    s_acc[cb] = s_acc[cb] + f
                    for d in range(D_POS):
                        u_acc[d][cb] = u_acc[d][cb] + p[d] * f
            for cb in range(CB):
                abuf_v[row0 + q, pl.ds(cb * L, L)] = s_acc[cb]
            for d in range(D_POS):
                for cb in range(CB):
                    abuf_v[row0 + q, pl.ds(C_IN + d * C_IN + cb * L, L)] = u_acc[d][cb]

    _gather_start(0, 0, sem0)

    def _block(blk, _):
        def _pair(gg, _):
            g0 = blk * GF + 2 * gg
            _gather_wait(g0, 0, sem0)
            _gather_start(g0 + 1, 1, sem1)
            _compute_group(0, 2 * gg * GP)
            _gather_wait(g0 + 1, 1, sem1)
            _gather_start(g0 + 2, 0, sem0)
            _compute_group(1, (2 * gg + 1) * GP)
            return 0
        lax.fori_loop(0, GF // 2, _pair, 0)
        pltpu.sync_copy(abuf_v, a_hbm.at[pl.ds(base_p + blk * FLUSH, FLUSH)])
        return 0

    lax.fori_loop(0, G_PER // GF, _block, 0)
    _gather_wait(G_PER, 0, sem0)


def _sc_segment_sums(table, nbr_pad):
    mesh = plsc.VectorSubcoreMesh(core_axis_name="c", subcore_axis_name="s",
                                  num_cores=NC, num_subcores=NS)
    return pl.kernel(
        _sc_body,
        out_type=jax.ShapeDtypeStruct((N_PAD, AW), jnp.float32),
        mesh=mesh,
        compiler_params=pltpu.CompilerParams(use_tc_tiling_on_sc=False),
        scratch_types=[
            pltpu.VMEM((G_PER + 8, GP * K), jnp.int32),
            pltpu.VMEM((2, GP * K, ROW), jnp.float32),
            pltpu.VMEM((FLUSH, AW), jnp.float32),
            pltpu.SemaphoreType.DMA,
            pltpu.SemaphoreType.DMA,
        ],
    )(table, nbr_pad)


def _tc_body(a_ref, w_ref, th_ref, p_ref, b_ref, o_ref):
    a = a_ref[...]
    y = jnp.dot(a, w_ref[...], preferred_element_type=jnp.float32)
    z = jnp.dot(a[:, :C_IN], th_ref[...], preferred_element_type=jnp.float32)
    for d in range(D_POS):
        y = y - p_ref[:, d:d + 1] * z[:, d * C_OUT:(d + 1) * C_OUT]
    o_ref[...] = y + b_ref[0:1, :]


def _tc_combine(a, w_full, th_cat, p_pad, bias2):
    return pl.pallas_call(
        _tc_body,
        grid=(N_PAD // BN,),
        in_specs=[
            pl.BlockSpec((BN, AW), lambda i: (i, 0)),
            pl.BlockSpec((AW, C_OUT), lambda i: (0, 0)),
            pl.BlockSpec((C_IN, D_POS * C_OUT), lambda i: (0, 0)),
            pl.BlockSpec((BN, 128), lambda i: (i, 0)),
            pl.BlockSpec((8, C_OUT), lambda i: (0, 0)),
        ],
        out_specs=pl.BlockSpec((BN, C_OUT), lambda i: (i, 0)),
        out_shape=jax.ShapeDtypeStruct((N_PAD, C_OUT), jnp.float32),
    )(a, w_full, th_cat, p_pad, bias2)


def kernel(features, weight_theta, weight_bias, bias, neighborhood, positions):
    fT = features[0].T
    pT = positions[0].T
    table = jnp.concatenate([fT, jnp.repeat(pT, L, axis=1)], axis=1)
    nbr_groups = jnp.zeros((N_PAD // GP + 8, GP * K), jnp.int32)
    nbr_groups = nbr_groups.at[:N // GP].set(
        neighborhood[0].T.reshape(N // GP, GP * K))

    a = _sc_segment_sums(table, nbr_groups)

    w_full = jnp.concatenate([weight_bias, weight_theta.reshape(D_POS * C_IN, C_OUT)], axis=0)
    th_cat = jnp.transpose(weight_theta, (1, 0, 2)).reshape(C_IN, D_POS * C_OUT)
    p_pad = jnp.zeros((N_PAD, 128), jnp.float32).at[:N, :D_POS].set(pT)
    bias2 = jnp.broadcast_to(bias[None, :], (8, C_OUT))

    out_t = _tc_combine(a, w_full, th_cat, p_pad, bias2)
    return out_t[:N].T[None]


# speedup vs baseline: 665.1277x; 1.2229x over previous
import functools

import jax
import jax.numpy as jnp
from jax import lax
from jax.experimental import pallas as pl
from jax.experimental.pallas import tpu as pltpu
from jax.experimental.pallas import tpu_sc as plsc

B, C_IN, C_OUT, N, K, D_POS = 1, 128, 128, 10000, 32, 3
NC, NS, L = 2, 16, 16
NW = NC * NS
N_PAD = 10240
P_PER = N_PAD // NW
ROW = C_IN + 4 * L
CB = C_IN // L
AW = C_IN * (1 + D_POS)
GP = 128 // K
G_PER = P_PER // GP
FLUSH = 32
GF = FLUSH // GP
BN = 512


def _sc_body(table_hbm, nbr_hbm, a_hbm, nbr_v, rows_v, abuf_v, sem0, sem1):
    wid = lax.axis_index("s") * NC + lax.axis_index("c")
    base_g = wid * G_PER
    base_p = wid * P_PER
    pltpu.sync_copy(nbr_hbm.at[pl.ds(base_g, G_PER + 8)], nbr_v)

    def _gather_start(g, buf, sem):
        return pltpu.async_copy(table_hbm.at[nbr_v.at[g]], rows_v.at[buf], sem)

    def _gather_wait(g, buf, sem):
        pltpu.make_async_copy(table_hbm.at[nbr_v.at[g]], rows_v.at[buf], sem).wait()

    def _compute_group(buf, row0):
        for q in range(GP):
            zero = jnp.zeros((L,), jnp.float32)
            s_acc = [zero] * CB
            u_acc = [[zero] * CB for _ in range(D_POS)]
            for k in range(K):
                r = q * K + k
                p0, p1 = plsc.unpack(rows_v[buf, r, pl.ds(C_IN, 2 * L)],
                                     format=plsc.PackFormat.INTERLEAVED)
                p2, _ = plsc.unpack(rows_v[buf, r, pl.ds(C_IN + 2 * L, 2 * L)],
                                    format=plsc.PackFormat.INTERLEAVED)
                p = (p0, p1, p2)
                for j in range(CB // 2):
                    fa, fb = plsc.unpack(rows_v[buf, r, pl.ds(j * 2 * L, 2 * L)],
                                         format=plsc.PackFormat.INTERLEAVED)
                    for cb, f in ((2 * j, fa), (2 * j + 1, fb)):
                        s_acc[cb] = s_acc[cb] + f
                        for d in range(D_POS):
                            u_acc[d][cb] = u_acc[d][cb] + p[d] * f
            for cb in range(CB):
                abuf_v[row0 + q, pl.ds(cb * L, L)] = s_acc[cb]
            for d in range(D_POS):
                for cb in range(CB):
                    abuf_v[row0 + q, pl.ds(C_IN + d * C_IN + cb * L, L)] = u_acc[d][cb]

    _gather_start(0, 0, sem0)

    def _block(blk, _):
        def _pair(gg, _):
            g0 = blk * GF + 2 * gg
            _gather_wait(g0, 0, sem0)
            _gather_start(g0 + 1, 1, sem1)
            _compute_group(0, 2 * gg * GP)
            _gather_wait(g0 + 1, 1, sem1)
            _gather_start(g0 + 2, 0, sem0)
            _compute_group(1, (2 * gg + 1) * GP)
            return 0
        lax.fori_loop(0, GF // 2, _pair, 0)
        pltpu.sync_copy(abuf_v, a_hbm.at[pl.ds(base_p + blk * FLUSH, FLUSH)])
        return 0

    lax.fori_loop(0, G_PER // GF, _block, 0)
    _gather_wait(G_PER, 0, sem0)


def _sc_segment_sums(table, nbr_pad):
    mesh = plsc.VectorSubcoreMesh(core_axis_name="c", subcore_axis_name="s",
                                  num_cores=NC, num_subcores=NS)
    return pl.kernel(
        _sc_body,
        out_type=jax.ShapeDtypeStruct((N_PAD, AW), jnp.float32),
        mesh=mesh,
        compiler_params=pltpu.CompilerParams(use_tc_tiling_on_sc=False,
                                             needs_layout_passes=False),
        scratch_types=[
            pltpu.VMEM((G_PER + 8, GP * K), jnp.int32),
            pltpu.VMEM((2, GP * K, ROW), jnp.bfloat16),
            pltpu.VMEM((FLUSH, AW), jnp.float32),
            pltpu.SemaphoreType.DMA,
            pltpu.SemaphoreType.DMA,
        ],
    )(table, nbr_pad)


def _tc_body(a_ref, w_ref, th_ref, p_ref, b_ref, o_ref):
    a = a_ref[...]
    y = jnp.dot(a, w_ref[...], preferred_element_type=jnp.float32)
    z = jnp.dot(a[:, :C_IN], th_ref[...], preferred_element_type=jnp.float32)
    for d in range(D_POS):
        y = y - p_ref[:, d:d + 1] * z[:, d * C_OUT:(d + 1) * C_OUT]
    o_ref[...] = y + b_ref[0:1, :]


def _tc_combine(a, w_full, th_cat, p_pad, bias2):
    return pl.pallas_call(
        _tc_body,
        grid=(N_PAD // BN,),
        in_specs=[
            pl.BlockSpec((BN, AW), lambda i: (i, 0)),
            pl.BlockSpec((AW, C_OUT), lambda i: (0, 0)),
            pl.BlockSpec((C_IN, D_POS * C_OUT), lambda i: (0, 0)),
            pl.BlockSpec((BN, 128), lambda i: (i, 0)),
            pl.BlockSpec((8, C_OUT), lambda i: (0, 0)),
        ],
        out_specs=pl.BlockSpec((BN, C_OUT), lambda i: (i, 0)),
        out_shape=jax.ShapeDtypeStruct((N_PAD, C_OUT), jnp.float32),
    )(a, w_full, th_cat, p_pad, bias2)


def _interleave(a, b):
    return jnp.stack([a, b], axis=-1).reshape(a.shape[0], -1)


def kernel(features, weight_theta, weight_bias, bias, neighborhood, positions):
    fT = features[0].T
    pT = positions[0].T
    f_blocks = [_interleave(fT[:, 32 * j:32 * j + 16], fT[:, 32 * j + 16:32 * j + 32])
                for j in range(4)]
    p_rep = [jnp.broadcast_to(pT[:, d:d + 1], (N, L)) for d in range(D_POS)]
    sec0 = _interleave(p_rep[0], p_rep[1])
    sec1 = _interleave(p_rep[2], jnp.zeros((N, L), jnp.float32))
    table = jnp.concatenate(f_blocks + [sec0, sec1], axis=1).astype(jnp.bfloat16)

    nbr_groups = jnp.zeros((N_PAD // GP + 8, GP * K), jnp.int32)
    nbr_groups = nbr_groups.at[:N // GP].set(
        neighborhood[0].T.reshape(N // GP, GP * K))

    a = _sc_segment_sums(table, nbr_groups)

    w_full = jnp.concatenate([weight_bias, weight_theta.reshape(D_POS * C_IN, C_OUT)], axis=0)
    th_cat = jnp.transpose(weight_theta, (1, 0, 2)).reshape(C_IN, D_POS * C_OUT)
    p_pad = jnp.zeros((N_PAD, 128), jnp.float32).at[:N, :D_POS].set(pT)
    bias2 = jnp.broadcast_to(bias[None, :], (8, C_OUT))

    out_t = _tc_combine(a, w_full, th_cat, p_pad, bias2)
    return out_t[:N].T[None]


# speedup vs baseline: 1469.3154x; 2.2091x over previous
import functools

import jax
import jax.numpy as jnp
from jax import lax
from jax.experimental import pallas as pl
from jax.experimental.pallas import tpu as pltpu
from jax.experimental.pallas import tpu_sc as plsc

B, C_IN, C_OUT, N, K, D_POS = 1, 128, 128, 10000, 32, 3
NC, NS, L = 2, 16, 16
N_PAD = 10240
P_SC = N_PAD // NC
CPT = C_IN // NS
ACT = CPT * (1 + D_POS)
PBLK = 128
NCHUNK = P_SC // PBLK
BN = 512


def _sc_body(f_hbm, p_hbm, nbr_hbm, a_hbm, f_v, p_v, nbr_v, abuf_v):
    sc = lax.axis_index("c")
    tid = lax.axis_index("s")
    pltpu.sync_copy(f_hbm.at[pl.ds(tid * CPT, CPT)], f_v)
    pltpu.sync_copy(p_hbm, p_v)
    pt_base = sc * P_SC

    cidx = [jnp.full((L,), c, jnp.int32) for c in range(CPT)]
    didx = [jnp.full((L,), d, jnp.int32) for d in range(D_POS)]

    def _block16(b16):
        zero = jnp.zeros((L,), jnp.float32)
        s_acc = [zero] * CPT
        u_acc = [[zero] * CPT for _ in range(D_POS)]
        for k in range(K):
            jk = nbr_v[k, pl.ds(b16 * L, L)]
            p = [plsc.load_gather(p_v, [didx[d], jk]) for d in range(D_POS)]
            for c in range(CPT):
                f = plsc.load_gather(f_v, [cidx[c], jk])
                s_acc[c] = s_acc[c] + f
                for d in range(D_POS):
                    u_acc[d][c] = u_acc[d][c] + p[d] * f
        for c in range(CPT):
            abuf_v[c, pl.ds(b16 * L, L)] = s_acc[c]
            for d in range(D_POS):
                abuf_v[CPT + d * CPT + c, pl.ds(b16 * L, L)] = u_acc[d][c]

    def _chunk(ch, _):
        pt0 = pt_base + ch * PBLK
        pltpu.sync_copy(nbr_hbm.at[:, pl.ds(pt0, PBLK)], nbr_v)

        def _blk(b16, _):
            _block16(b16)
            return 0
        lax.fori_loop(0, PBLK // L, _blk, 0)
        pltpu.sync_copy(abuf_v.at[pl.ds(0, CPT)],
                        a_hbm.at[pl.ds(tid * CPT, CPT), pl.ds(pt0, PBLK)])
        pltpu.sync_copy(abuf_v.at[pl.ds(CPT, D_POS * CPT)],
                        a_hbm.at[pl.ds(C_IN + tid * D_POS * CPT, D_POS * CPT),
                                 pl.ds(pt0, PBLK)])
        return 0

    lax.fori_loop(0, NCHUNK, _chunk, 0)


def _sc_segment_sums(f, p, nbr_pad):
    mesh = plsc.VectorSubcoreMesh(core_axis_name="c", subcore_axis_name="s",
                                  num_cores=NC, num_subcores=NS)
    return pl.kernel(
        _sc_body,
        out_type=jax.ShapeDtypeStruct((C_IN * (1 + D_POS), N_PAD), jnp.float32),
        mesh=mesh,
        compiler_params=pltpu.CompilerParams(use_tc_tiling_on_sc=False,
                                             needs_layout_passes=False),
        scratch_types=[
            pltpu.VMEM((CPT, N), jnp.float32),
            pltpu.VMEM((D_POS, N), jnp.float32),
            pltpu.VMEM((K, PBLK), jnp.int32),
            pltpu.VMEM((ACT, PBLK), jnp.float32),
        ],
    )(f, p, nbr_pad)


def _tc_body(a_ref, w_ref, th_ref, p_ref, b_ref, o_ref):
    a = a_ref[...]
    y = jnp.dot(w_ref[...], a, preferred_element_type=jnp.float32)
    z = jnp.dot(th_ref[...], a[:C_IN, :], preferred_element_type=jnp.float32)
    for d in range(D_POS):
        y = y - p_ref[d:d + 1, :] * z[d * C_OUT:(d + 1) * C_OUT, :]
    o_ref[...] = y + b_ref[:, 0:1]


def _tc_combine(a, w_all_t, th_t, p_pad, bias_col):
    return pl.pallas_call(
        _tc_body,
        grid=(N_PAD // BN,),
        in_specs=[
            pl.BlockSpec((C_IN * (1 + D_POS), BN), lambda i: (0, i)),
            pl.BlockSpec((C_OUT, C_IN * (1 + D_POS)), lambda i: (0, 0)),
            pl.BlockSpec((D_POS * C_OUT, C_IN), lambda i: (0, 0)),
            pl.BlockSpec((8, BN), lambda i: (0, i)),
            pl.BlockSpec((C_OUT, 8), lambda i: (0, 0)),
        ],
        out_specs=pl.BlockSpec((C_OUT, BN), lambda i: (0, i)),
        out_shape=jax.ShapeDtypeStruct((C_OUT, N_PAD), jnp.float32),
    )(a, w_all_t, th_t, p_pad, bias_col)


def kernel(features, weight_theta, weight_bias, bias, neighborhood, positions):
    f = features[0]
    p = positions[0]
    nbr_pad = jnp.zeros((K, N_PAD), jnp.int32).at[:, :N].set(neighborhood[0])

    a = _sc_segment_sums(f, p, nbr_pad)

    th_u = jnp.transpose(weight_theta, (1, 0, 2)).reshape(NS, CPT, D_POS, C_OUT)
    th_u = jnp.transpose(th_u, (0, 2, 1, 3)).reshape(D_POS * C_IN, C_OUT)
    w_all_t = jnp.concatenate([weight_bias, th_u], axis=0).T
    th_t = jnp.transpose(weight_theta, (0, 2, 1)).reshape(D_POS * C_OUT, C_IN)
    p_pad = jnp.zeros((8, N_PAD), jnp.float32).at[:D_POS, :N].set(p)
    bias_col = jnp.broadcast_to(bias[:, None], (C_OUT, 8))

    out_t = _tc_combine(a, w_all_t, th_t, p_pad, bias_col)
    return out_t[:, :N][None]


# speedup vs baseline: 1676.3839x; 1.1409x over previous
import functools

import jax
import jax.numpy as jnp
from jax import lax
from jax.experimental import pallas as pl
from jax.experimental.pallas import tpu as pltpu
from jax.experimental.pallas import tpu_sc as plsc

B, C_IN, C_OUT, N, K, D_POS = 1, 128, 128, 10000, 32, 3
NC, NS, L = 2, 16, 16
N_PAD = 10240
P_SC = N_PAD // NC
CPT = C_IN // NS
ACT = CPT * (1 + D_POS)
PBLK = 128
NCHUNK = P_SC // PBLK
BN = 512


def _sc_body(f_hbm, p_hbm, nbr_hbm, a_hbm, f_v, p_v, nbr_v, abuf_v,
             semn0, semn1, semf0, semf1):
    sc = lax.axis_index("c")
    tid = lax.axis_index("s")
    pltpu.sync_copy(f_hbm.at[pl.ds(tid * CPT, CPT)], f_v)
    pltpu.sync_copy(p_hbm, p_v)
    pt_base = sc * P_SC
    semn = (semn0, semn1)
    semf = (semf0, semf1)

    cidx = [jnp.full((L,), c, jnp.int32) for c in range(CPT)]
    didx = [jnp.full((L,), d, jnp.int32) for d in range(D_POS)]

    def _nbr_start(ch, buf):
        pltpu.async_copy(nbr_hbm.at[:, pl.ds(pt_base + ch * PBLK, PBLK)],
                         nbr_v.at[buf], semn[buf])

    def _nbr_wait(ch, buf):
        pltpu.make_async_copy(nbr_hbm.at[:, pl.ds(pt_base + ch * PBLK, PBLK)],
                              nbr_v.at[buf], semn[buf]).wait()

    def _flush_descs(ch, buf):
        pt0 = pt_base + ch * PBLK
        d0 = pltpu.make_async_copy(
            abuf_v.at[buf, pl.ds(0, CPT)],
            a_hbm.at[pl.ds(tid * CPT, CPT), pl.ds(pt0, PBLK)], semf[buf])
        d1 = pltpu.make_async_copy(
            abuf_v.at[buf, pl.ds(CPT, D_POS * CPT)],
            a_hbm.at[pl.ds(C_IN + tid * D_POS * CPT, D_POS * CPT),
                     pl.ds(pt0, PBLK)], semf[buf])
        return d0, d1

    def _block16(nbuf, b16):
        zero = jnp.zeros((L,), jnp.float32)
        s_acc = [zero] * CPT
        u_acc = [[zero] * CPT for _ in range(D_POS)]
        for k in range(K):
            jk = nbr_v[nbuf, k, pl.ds(b16 * L, L)]
            p = [plsc.load_gather(p_v, [didx[d], jk]) for d in range(D_POS)]
            for c in range(CPT):
                f = plsc.load_gather(f_v, [cidx[c], jk])
                s_acc[c] = s_acc[c] + f
                for d in range(D_POS):
                    u_acc[d][c] = u_acc[d][c] + p[d] * f
        for c in range(CPT):
            abuf_v[nbuf, c, pl.ds(b16 * L, L)] = s_acc[c]
            for d in range(D_POS):
                abuf_v[nbuf, CPT + d * CPT + c, pl.ds(b16 * L, L)] = u_acc[d][c]

    def _compute_chunk(ch, buf):
        _nbr_wait(ch, buf)
        _nbr_start(ch + 2, buf)

        def _blk(b16, _):
            _block16(buf, b16)
            return 0
        lax.fori_loop(0, PBLK // L, _blk, 0)
        d0, d1 = _flush_descs(ch, buf)
        d0.start()
        d1.start()

    _nbr_start(0, 0)
    _nbr_start(1, 1)

    def _pair(j, _):
        ch = 2 * j

        @pl.when(j > 0)
        def _():
            da, db = _flush_descs(2 * j - 2, 0)
            da.wait()
            db.wait()
        _compute_chunk(ch, 0)

        @pl.when(j > 0)
        def _():
            da, db = _flush_descs(2 * j - 1, 1)
            da.wait()
            db.wait()
        _compute_chunk(ch + 1, 1)
        return 0

    lax.fori_loop(0, NCHUNK // 2, _pair, 0)
    da, db = _flush_descs(NCHUNK - 2, 0)
    da.wait()
    db.wait()
    da, db = _flush_descs(NCHUNK - 1, 1)
    da.wait()
    db.wait()
    _nbr_wait(NCHUNK, 0)
    _nbr_wait(NCHUNK + 1, 1)


def _sc_segment_sums(f, p, nbr_pad):
    mesh = plsc.VectorSubcoreMesh(core_axis_name="c", subcore_axis_name="s",
                                  num_cores=NC, num_subcores=NS)
    return pl.kernel(
        _sc_body,
        out_type=jax.ShapeDtypeStruct((C_IN * (1 + D_POS), N_PAD), jnp.float32),
        mesh=mesh,
        compiler_params=pltpu.CompilerParams(use_tc_tiling_on_sc=False,
                                             needs_layout_passes=False),
        scratch_types=[
            pltpu.VMEM((CPT, N), jnp.float32),
            pltpu.VMEM((D_POS, N), jnp.float32),
            pltpu.VMEM((2, K, PBLK), jnp.int32),
            pltpu.VMEM((2, ACT, PBLK), jnp.float32),
            pltpu.SemaphoreType.DMA,
            pltpu.SemaphoreType.DMA,
            pltpu.SemaphoreType.DMA,
            pltpu.SemaphoreType.DMA,
        ],
    )(f, p, nbr_pad)


def _tc_body(a_ref, w_ref, th_ref, p_ref, b_ref, o_ref):
    a = a_ref[...]
    y = jnp.dot(w_ref[...], a, preferred_element_type=jnp.float32)
    z = jnp.dot(th_ref[...], a[:C_IN, :], preferred_element_type=jnp.float32)
    for d in range(D_POS):
        y = y - p_ref[d:d + 1, :] * z[d * C_OUT:(d + 1) * C_OUT, :]
    o_ref[...] = y + b_ref[:, 0:1]


def _tc_combine(a, w_all_t, th_t, p_pad, bias_col):
    return pl.pallas_call(
        _tc_body,
        grid=(N_PAD // BN,),
        in_specs=[
            pl.BlockSpec((C_IN * (1 + D_POS), BN), lambda i: (0, i)),
            pl.BlockSpec((C_OUT, C_IN * (1 + D_POS)), lambda i: (0, 0)),
            pl.BlockSpec((D_POS * C_OUT, C_IN), lambda i: (0, 0)),
            pl.BlockSpec((8, BN), lambda i: (0, i)),
            pl.BlockSpec((C_OUT, 8), lambda i: (0, 0)),
        ],
        out_specs=pl.BlockSpec((C_OUT, BN), lambda i: (0, i)),
        out_shape=jax.ShapeDtypeStruct((C_OUT, N_PAD), jnp.float32),
    )(a, w_all_t, th_t, p_pad, bias_col)


def kernel(features, weight_theta, weight_bias, bias, neighborhood, positions):
    f = features[0]
    p = positions[0]
    nbr_pad = jnp.zeros((K, N_PAD + 2 * PBLK), jnp.int32).at[:, :N].set(neighborhood[0])

    a = _sc_segment_sums(f, p, nbr_pad)

    th_u = jnp.transpose(weight_theta, (1, 0, 2)).reshape(NS, CPT, D_POS, C_OUT)
    th_u = jnp.transpose(th_u, (0, 2, 1, 3)).reshape(D_POS * C_IN, C_OUT)
    w_all_t = jnp.concatenate([weight_bias, th_u], axis=0).T
    th_t = jnp.transpose(weight_theta, (0, 2, 1)).reshape(D_POS * C_OUT, C_IN)
    p_pad = jnp.zeros((8, N_PAD), jnp.float32).at[:D_POS, :N].set(p)
    bias_col = jnp.broadcast_to(bias[:, None], (C_OUT, 8))

    out_t = _tc_combine(a, w_all_t, th_t, p_pad, bias_col)
    return out_t[:, :N][None]
